# Initial kernel scaffold; baseline (speedup 1.0000x reference)
import functools

import jax
import jax.numpy as jnp
from jax import lax
from jax.experimental import pallas as pl
from jax.experimental.pallas import tpu as pltpu
from jax.experimental.pallas import tpu_sc as plsc

N = 10000
E = 320000
D_IN = 128
D_HID = 128
D_OUT = 64

NC = 2
NS = 16
CHUNK = 125
TILE_ROWS = (E // NS) // CHUNK
ROWS_TOTAL = 2 * E // CHUNK
NODES_PER_TILE = N // NS
DEG_W = 16
DEG_ZROWS = 2 * N // NS

_mesh = plsc.VectorSubcoreMesh(core_axis_name="c", subcore_axis_name="s")


@functools.partial(
    pl.kernel,
    out_type=jax.ShapeDtypeStruct((2, 2 * N, DEG_W), jnp.float32),
    mesh=_mesh,
    scratch_types=[
        pltpu.VMEM_SHARED((2 * N, DEG_W), jnp.float32),
        pltpu.VMEM_SHARED((2 * N, DEG_W), jnp.float32),
        pltpu.VMEM((TILE_ROWS, CHUNK), jnp.int32),
        pltpu.VMEM((TILE_ROWS, CHUNK), jnp.int32),
        pltpu.VMEM((CHUNK, DEG_W), jnp.float32),
    ],
)
def _deg_kernel(src_hbm, dst_hbm, ones_hbm, zeros_hbm, cnt_hbm,
                acc_out, acc_in, sidx, didx, ones_v):
    c = lax.axis_index("c")
    s = lax.axis_index("s")
    pltpu.sync_copy(zeros_hbm, acc_out.at[pl.ds(s * DEG_ZROWS, DEG_ZROWS)])
    pltpu.sync_copy(zeros_hbm, acc_in.at[pl.ds(s * DEG_ZROWS, DEG_ZROWS)])
    pltpu.sync_copy(ones_hbm, ones_v)
    base = c * (ROWS_TOTAL // 2) + s * TILE_ROWS
    pltpu.sync_copy(src_hbm.at[pl.ds(base, TILE_ROWS)], sidx)
    pltpu.sync_copy(dst_hbm.at[pl.ds(base, TILE_ROWS)], didx)
    plsc.subcore_barrier()

    def chunk(j, carry):
        pltpu.sync_copy(ones_v, acc_out.at[sidx.at[j]], add=True)
        pltpu.sync_copy(ones_v, acc_in.at[didx.at[j]], add=True)
        return carry

    lax.fori_loop(0, TILE_ROWS, chunk, 0)
    plsc.subcore_barrier()
    r0 = c * N + s * NODES_PER_TILE
    pltpu.sync_copy(acc_out.at[pl.ds(r0, NODES_PER_TILE)],
                    cnt_hbm.at[0, pl.ds(r0, NODES_PER_TILE)])
    pltpu.sync_copy(acc_in.at[pl.ds(r0, NODES_PER_TILE)],
                    cnt_hbm.at[1, pl.ds(r0, NODES_PER_TILE)])


def _make_spmm(d):
    @functools.partial(
        pl.kernel,
        out_type=jax.ShapeDtypeStruct((2 * N, d), jnp.float32),
        mesh=_mesh,
        scratch_types=[
            pltpu.VMEM_SHARED((N, d), jnp.float32),
            pltpu.VMEM((TILE_ROWS, CHUNK), jnp.int32),
            pltpu.VMEM((TILE_ROWS, CHUNK), jnp.int32),
            pltpu.VMEM((CHUNK, d), jnp.float32),
            pltpu.SemaphoreType.DMA,
        ],
    )
    def spmm(src_hbm, dst_hbm, g_hbm, zeros_hbm, out_hbm,
             acc, sidx, didx, rows, gsem):
        c = lax.axis_index("c")
        s = lax.axis_index("s")
        pltpu.sync_copy(zeros_hbm,
                        acc.at[pl.ds(s * NODES_PER_TILE, NODES_PER_TILE)])
        base = c * (ROWS_TOTAL // 2) + s * TILE_ROWS
        pltpu.sync_copy(src_hbm.at[pl.ds(base, TILE_ROWS)], sidx)
        pltpu.sync_copy(dst_hbm.at[pl.ds(base, TILE_ROWS)], didx)
        plsc.subcore_barrier()

        def chunk(j, carry):
            pltpu.async_copy(g_hbm.at[sidx.at[j]], rows, gsem).wait()
            pltpu.sync_copy(rows, acc.at[didx.at[j]], add=True)
            return carry

        lax.fori_loop(0, TILE_ROWS, chunk, 0)
        plsc.subcore_barrier()
        r0 = s * NODES_PER_TILE
        pltpu.sync_copy(acc.at[pl.ds(r0, NODES_PER_TILE)],
                        out_hbm.at[pl.ds(c * N + r0, NODES_PER_TILE)])

    return spmm


_spmm_hid = _make_spmm(D_HID)
_spmm_out = _make_spmm(D_OUT)


def _rsqrt_deg(cnt_block):
    return lax.rsqrt(jnp.maximum(cnt_block[0, :, :1], 1.0))


RB = 1000


def _mm1_body(x_ref, w_ref, cnt_ref, o_ref):
    r = _rsqrt_deg(cnt_ref[...])
    o_ref[...] = jax.lax.dot(
        x_ref[...], w_ref[0], preferred_element_type=jnp.float32) * r


def _mm1(X, W1s, cnts):
    return pl.pallas_call(
        _mm1_body,
        grid=(2, N // RB),
        in_specs=[
            pl.BlockSpec((RB, D_IN), lambda v, i: (i, 0)),
            pl.BlockSpec((1, D_IN, D_HID), lambda v, i: (v, 0, 0)),
            pl.BlockSpec((1, RB, DEG_W), lambda v, i: (0, v * (N // RB) + i, 0)),
        ],
        out_specs=pl.BlockSpec((RB, D_HID), lambda v, i: (v * (N // RB) + i, 0)),
        out_shape=jax.ShapeDtypeStruct((2 * N, D_HID), jnp.float32),
    )(X, W1s, cnts)


def _mid_body(a0_ref, a1_ref, ci0_ref, ci1_ref, b10_ref, b11_ref,
              w2_ref, co_ref, o_ref):
    h0 = jnp.maximum(a0_ref[...] * _rsqrt_deg(ci0_ref[...]) + b10_ref[...], 0.0)
    h1 = jnp.maximum(a1_ref[...] * _rsqrt_deg(ci1_ref[...]) + b11_ref[...], 0.0)
    h = (h0 + h1) * 0.5
    o_ref[...] = jax.lax.dot(
        h, w2_ref[0], preferred_element_type=jnp.float32
    ) * _rsqrt_deg(co_ref[...])


def _mid(agg1, cnts, b1s, W2s):
    nb = N // RB
    return pl.pallas_call(
        _mid_body,
        grid=(2, nb),
        in_specs=[
            pl.BlockSpec((RB, D_HID), lambda v, i: (i, 0)),
            pl.BlockSpec((RB, D_HID), lambda v, i: (nb + i, 0)),
            pl.BlockSpec((1, RB, DEG_W), lambda v, i: (1, i, 0)),
            pl.BlockSpec((1, RB, DEG_W), lambda v, i: (1, nb + i, 0)),
            pl.BlockSpec((1, D_HID), lambda v, i: (0, 0)),
            pl.BlockSpec((1, D_HID), lambda v, i: (1, 0)),
            pl.BlockSpec((1, D_HID, D_OUT), lambda v, i: (v, 0, 0)),
            pl.BlockSpec((1, RB, DEG_W), lambda v, i: (0, v * nb + i, 0)),
        ],
        out_specs=pl.BlockSpec((RB, D_OUT), lambda v, i: (v * nb + i, 0)),
        out_shape=jax.ShapeDtypeStruct((2 * N, D_OUT), jnp.float32),
    )(agg1, agg1, cnts, cnts, b1s, b1s, W2s, cnts)


def _fin_body(a0_ref, a1_ref, ci0_ref, ci1_ref, b20_ref, b21_ref, o_ref):
    y0 = a0_ref[...] * _rsqrt_deg(ci0_ref[...]) + b20_ref[...]
    y1 = a1_ref[...] * _rsqrt_deg(ci1_ref[...]) + b21_ref[...]
    o_ref[...] = (y0 + y1) * 0.5


def _fin(agg2, cnts, b2s):
    nb = N // RB
    return pl.pallas_call(
        _fin_body,
        grid=(nb,),
        in_specs=[
            pl.BlockSpec((RB, D_OUT), lambda i: (i, 0)),
            pl.BlockSpec((RB, D_OUT), lambda i: (nb + i, 0)),
            pl.BlockSpec((1, RB, DEG_W), lambda i: (1, i, 0)),
            pl.BlockSpec((1, RB, DEG_W), lambda i: (1, nb + i, 0)),
            pl.BlockSpec((1, D_OUT), lambda i: (0, 0)),
            pl.BlockSpec((1, D_OUT), lambda i: (1, 0)),
        ],
        out_specs=pl.BlockSpec((RB, D_OUT), lambda i: (i, 0)),
        out_shape=jax.ShapeDtypeStruct((N, D_OUT), jnp.float32),
    )(agg2, agg2, cnts, cnts, b2s, b2s)


@jax.jit
def kernel(X, edge_index_v0, edge_index_v1, W1_v0, b1_v0, W1_v1, b1_v1,
           W2_v0, b2_v0, W2_v1, b2_v1):
    s0, d0 = edge_index_v0[0], edge_index_v0[1]
    s1, d1 = edge_index_v1[0], edge_index_v1[1]
    src_off = jnp.concatenate([s0, s1 + N]).reshape(ROWS_TOTAL, CHUNK)
    dst_off = jnp.concatenate([d0, d1 + N]).reshape(ROWS_TOTAL, CHUNK)
    dst_raw = jnp.concatenate([d0, d1]).reshape(ROWS_TOTAL, CHUNK)

    ones_deg = jnp.ones((CHUNK, DEG_W), jnp.float32)
    zeros_deg = jnp.zeros((DEG_ZROWS, DEG_W), jnp.float32)
    zeros_hid = jnp.zeros((NODES_PER_TILE, D_HID), jnp.float32)
    zeros_out = jnp.zeros((NODES_PER_TILE, D_OUT), jnp.float32)

    W1s = jnp.stack([W1_v0, W1_v1])
    W2s = jnp.stack([W2_v0, W2_v1])
    b1s = jnp.stack([b1_v0, b1_v1])
    b2s = jnp.stack([b2_v0, b2_v1])

    cnts = _deg_kernel(src_off, dst_off, ones_deg, zeros_deg)
    g1 = _mm1(X, W1s, cnts)
    agg1 = _spmm_hid(src_off, dst_raw, g1, zeros_hid)
    g2 = _mid(agg1, cnts, b1s, W2s)
    agg2 = _spmm_out(src_off, dst_raw, g2, zeros_out)
    return _fin(agg2, cnts, b2s)


# speedup vs baseline: 5.0537x; 5.0537x over previous
import functools

import jax
import jax.numpy as jnp
from jax import lax
from jax.experimental import pallas as pl
from jax.experimental.pallas import tpu as pltpu
from jax.experimental.pallas import tpu_sc as plsc

N = 10000
E = 320000
D_IN = 128
D_HID = 128
D_OUT = 64

NC = 2
NS = 16
CHUNK = 128
ROWS_VIEW = -(-E // (NS * CHUNK)) * NS
E_PAD = ROWS_VIEW * CHUNK
TILE_ROWS = ROWS_VIEW // NS
DEG_W = 16

SLAB = 624
SLAB_LAST = N - SLAB * (NS - 1)
SLAB_LAST_OFS = SLAB * (NS - 1)
DSLAB = 1248
DSLAB_LAST = 2 * N - DSLAB * (NS - 1)
DSLAB_LAST_OFS = DSLAB * (NS - 1)


@functools.cache
def _sc_kernels():
    mesh = plsc.VectorSubcoreMesh(
        core_axis_name="c", subcore_axis_name="s",
        num_cores=NC, num_subcores=NS)
    params = pltpu.CompilerParams(use_tc_tiling_on_sc=False)

    @functools.partial(
        pl.kernel,
        out_type=jax.ShapeDtypeStruct((2, 2 * N, DEG_W), jnp.float32),
        mesh=mesh,
        compiler_params=params,
        scratch_types=[
            pltpu.VMEM_SHARED((2 * N + 8, DEG_W), jnp.float32),
            pltpu.VMEM_SHARED((2 * N + 8, DEG_W), jnp.float32),
            pltpu.VMEM((CHUNK,), jnp.int32),
            pltpu.VMEM((CHUNK,), jnp.int32),
            pltpu.VMEM((CHUNK, DEG_W), jnp.float32),
        ],
    )
    def deg_kernel(src_hbm, dst_hbm, ones_hbm, zeros_hbm, cnt_hbm,
                   acc_out, acc_in, sidx, didx, ones_v):
        c = lax.axis_index("c")
        s = lax.axis_index("s")

        @pl.when(s < NS - 1)
        def _():
            pltpu.sync_copy(zeros_hbm.at[pl.ds(0, DSLAB)],
                            acc_out.at[pl.ds(s * DSLAB, DSLAB)])
            pltpu.sync_copy(zeros_hbm.at[pl.ds(0, DSLAB)],
                            acc_in.at[pl.ds(s * DSLAB, DSLAB)])

        @pl.when(s == NS - 1)
        def _():
            pltpu.sync_copy(zeros_hbm,
                            acc_out.at[pl.ds(DSLAB_LAST_OFS, DSLAB_LAST)])
            pltpu.sync_copy(zeros_hbm,
                            acc_in.at[pl.ds(DSLAB_LAST_OFS, DSLAB_LAST)])

        pltpu.sync_copy(ones_hbm, ones_v)
        base = (c * ROWS_VIEW + s * TILE_ROWS) * CHUNK
        plsc.subcore_barrier()

        def chunk(j, carry):
            off = base + j * CHUNK
            pltpu.sync_copy(src_hbm.at[pl.ds(off, CHUNK)], sidx)
            pltpu.sync_copy(dst_hbm.at[pl.ds(off, CHUNK)], didx)
            pltpu.sync_copy(ones_v, acc_out.at[sidx], add=True)
            pltpu.sync_copy(ones_v, acc_in.at[didx], add=True)
            return carry

        lax.fori_loop(0, TILE_ROWS, chunk, 0)
        plsc.subcore_barrier()

        @pl.when(s < NS - 1)
        def _():
            r0 = c * N + s * SLAB
            pltpu.sync_copy(acc_out.at[pl.ds(r0, SLAB)],
                            cnt_hbm.at[0, pl.ds(r0, SLAB)])
            pltpu.sync_copy(acc_in.at[pl.ds(r0, SLAB)],
                            cnt_hbm.at[1, pl.ds(r0, SLAB)])

        @pl.when(s == NS - 1)
        def _():
            r0 = c * N + SLAB_LAST_OFS
            pltpu.sync_copy(acc_out.at[pl.ds(r0, SLAB_LAST)],
                            cnt_hbm.at[0, pl.ds(r0, SLAB_LAST)])
            pltpu.sync_copy(acc_in.at[pl.ds(r0, SLAB_LAST)],
                            cnt_hbm.at[1, pl.ds(r0, SLAB_LAST)])

    def make_spmm(d):
        @functools.partial(
            pl.kernel,
            out_type=jax.ShapeDtypeStruct((2 * N, d), jnp.float32),
            mesh=mesh,
            compiler_params=params,
            scratch_types=[
                pltpu.VMEM_SHARED((N + 8, d), jnp.float32),
                pltpu.VMEM((CHUNK,), jnp.int32),
                pltpu.VMEM((CHUNK,), jnp.int32),
                pltpu.VMEM((CHUNK, d), jnp.float32),
                pltpu.SemaphoreType.DMA,
            ],
        )
        def spmm(src_hbm, dst_hbm, g_hbm, zeros_hbm, out_hbm,
                 acc, sidx, didx, rows, gsem):
            c = lax.axis_index("c")
            s = lax.axis_index("s")

            @pl.when(s < NS - 1)
            def _():
                pltpu.sync_copy(zeros_hbm.at[pl.ds(0, SLAB)],
                                acc.at[pl.ds(s * SLAB, SLAB)])

            @pl.when(s == NS - 1)
            def _():
                pltpu.sync_copy(zeros_hbm,
                                acc.at[pl.ds(SLAB_LAST_OFS, SLAB_LAST)])

            base = (c * ROWS_VIEW + s * TILE_ROWS) * CHUNK
            plsc.subcore_barrier()

            def chunk(j, carry):
                off = base + j * CHUNK
                pltpu.sync_copy(src_hbm.at[pl.ds(off, CHUNK)], sidx)
                pltpu.sync_copy(dst_hbm.at[pl.ds(off, CHUNK)], didx)
                pltpu.async_copy(g_hbm.at[sidx], rows, gsem).wait()
                pltpu.sync_copy(rows, acc.at[didx], add=True)
                return carry

            lax.fori_loop(0, TILE_ROWS, chunk, 0)
            plsc.subcore_barrier()

            @pl.when(s < NS - 1)
            def _():
                r0 = s * SLAB
                pltpu.sync_copy(acc.at[pl.ds(r0, SLAB)],
                                out_hbm.at[pl.ds(c * N + r0, SLAB)])

            @pl.when(s == NS - 1)
            def _():
                pltpu.sync_copy(
                    acc.at[pl.ds(SLAB_LAST_OFS, SLAB_LAST)],
                    out_hbm.at[pl.ds(c * N + SLAB_LAST_OFS, SLAB_LAST)])

        return spmm

    return deg_kernel, make_spmm(D_HID), make_spmm(D_OUT)


def _rsqrt_deg(cnt_block):
    return lax.rsqrt(jnp.maximum(cnt_block[0, :, :1], 1.0))


RB = 1000


def _mm1_body(x_ref, w_ref, cnt_ref, o_ref):
    r = _rsqrt_deg(cnt_ref[...])
    o_ref[...] = jax.lax.dot(
        x_ref[...], w_ref[0], preferred_element_type=jnp.float32) * r


def _mm1(X, W1s, cnts):
    return pl.pallas_call(
        _mm1_body,
        grid=(2, N // RB),
        in_specs=[
            pl.BlockSpec((RB, D_IN), lambda v, i: (i, 0)),
            pl.BlockSpec((1, D_IN, D_HID), lambda v, i: (v, 0, 0)),
            pl.BlockSpec((1, RB, DEG_W), lambda v, i: (0, v * (N // RB) + i, 0)),
        ],
        out_specs=pl.BlockSpec((RB, D_HID), lambda v, i: (v * (N // RB) + i, 0)),
        out_shape=jax.ShapeDtypeStruct((2 * N, D_HID), jnp.float32),
    )(X, W1s, cnts)


def _mid_body(a0_ref, a1_ref, ci0_ref, ci1_ref, b1s_ref,
              w2_ref, co_ref, o_ref):
    h0 = jnp.maximum(
        a0_ref[...] * _rsqrt_deg(ci0_ref[...]) + b1s_ref[0][None], 0.0)
    h1 = jnp.maximum(
        a1_ref[...] * _rsqrt_deg(ci1_ref[...]) + b1s_ref[1][None], 0.0)
    h = (h0 + h1) * 0.5
    o_ref[...] = jax.lax.dot(
        h, w2_ref[0], preferred_element_type=jnp.float32
    ) * _rsqrt_deg(co_ref[...])


def _mid(agg1, cnts, b1s, W2s):
    nb = N // RB
    return pl.pallas_call(
        _mid_body,
        grid=(2, nb),
        in_specs=[
            pl.BlockSpec((RB, D_HID), lambda v, i: (i, 0)),
            pl.BlockSpec((RB, D_HID), lambda v, i: (nb + i, 0)),
            pl.BlockSpec((1, RB, DEG_W), lambda v, i: (1, i, 0)),
            pl.BlockSpec((1, RB, DEG_W), lambda v, i: (1, nb + i, 0)),
            pl.BlockSpec((2, D_HID), lambda v, i: (0, 0)),
            pl.BlockSpec((1, D_HID, D_OUT), lambda v, i: (v, 0, 0)),
            pl.BlockSpec((1, RB, DEG_W), lambda v, i: (0, v * nb + i, 0)),
        ],
        out_specs=pl.BlockSpec((RB, D_OUT), lambda v, i: (v * nb + i, 0)),
        out_shape=jax.ShapeDtypeStruct((2 * N, D_OUT), jnp.float32),
    )(agg1, agg1, cnts, cnts, b1s, W2s, cnts)


def _fin_body(a0_ref, a1_ref, ci0_ref, ci1_ref, b2s_ref, o_ref):
    y0 = a0_ref[...] * _rsqrt_deg(ci0_ref[...]) + b2s_ref[0][None]
    y1 = a1_ref[...] * _rsqrt_deg(ci1_ref[...]) + b2s_ref[1][None]
    o_ref[...] = (y0 + y1) * 0.5


def _fin(agg2, cnts, b2s):
    nb = N // RB
    return pl.pallas_call(
        _fin_body,
        grid=(nb,),
        in_specs=[
            pl.BlockSpec((RB, D_OUT), lambda i: (i, 0)),
            pl.BlockSpec((RB, D_OUT), lambda i: (nb + i, 0)),
            pl.BlockSpec((1, RB, DEG_W), lambda i: (1, i, 0)),
            pl.BlockSpec((1, RB, DEG_W), lambda i: (1, nb + i, 0)),
            pl.BlockSpec((2, D_OUT), lambda i: (0, 0)),
        ],
        out_specs=pl.BlockSpec((RB, D_OUT), lambda i: (i, 0)),
        out_shape=jax.ShapeDtypeStruct((N, D_OUT), jnp.float32),
    )(agg2, agg2, cnts, cnts, b2s)


@jax.jit
def kernel(X, edge_index_v0, edge_index_v1, W1_v0, b1_v0, W1_v1, b1_v1,
           W2_v0, b2_v0, W2_v1, b2_v1):
    s0, d0 = edge_index_v0[0], edge_index_v0[1]
    s1, d1 = edge_index_v1[0], edge_index_v1[1]
    npad = E_PAD - E
    pad_deg = jnp.full((npad,), 2 * N, jnp.int32)
    pad_gat = jnp.zeros((npad,), jnp.int32)
    pad_sct = jnp.full((npad,), N, jnp.int32)
    src_deg = jnp.concatenate([s0, pad_deg, s1 + N, pad_deg])
    dst_deg = jnp.concatenate([d0, pad_deg, d1 + N, pad_deg])
    src_gat = jnp.concatenate([s0, pad_gat, s1 + N, pad_gat])
    dst_sct = jnp.concatenate([d0, pad_sct, d1, pad_sct])

    ones_deg = jnp.ones((CHUNK, DEG_W), jnp.float32)
    zeros_deg = jnp.zeros((DSLAB_LAST, DEG_W), jnp.float32)
    zeros_hid = jnp.zeros((SLAB_LAST, D_HID), jnp.float32)
    zeros_out = jnp.zeros((SLAB_LAST, D_OUT), jnp.float32)

    W1s = jnp.stack([W1_v0, W1_v1])
    W2s = jnp.stack([W2_v0, W2_v1])
    b1s = jnp.stack([b1_v0, b1_v1])
    b2s = jnp.stack([b2_v0, b2_v1])

    deg_kernel, spmm_hid, spmm_out = _sc_kernels()
    cnts = deg_kernel(src_deg, dst_deg, ones_deg, zeros_deg)
    g1 = _mm1(X, W1s, cnts)
    agg1 = spmm_hid(src_gat, dst_sct, g1, zeros_hid)
    g2 = _mid(agg1, cnts, b1s, W2s)
    agg2 = spmm_out(src_gat, dst_sct, g2, zeros_out)
    return _fin(agg2, cnts, b2s)
